# baseline (speedup 1.0000x reference)
import functools

import jax
import jax.numpy as jnp
from jax import lax
from jax.experimental import pallas as pl
from jax.experimental.pallas import tpu as pltpu
from jax.experimental.pallas import tpu_sc as plsc

_NUM_ROWS = 16384
_NUM_COLS = 26
_DIM = 64
_NC = 2
_NS = 16
_NW = _NC * _NS
_IPW = _NUM_ROWS // _NW
_CHUNK = 256
_CPI = _IPW // _CHUNK
_NSTEP = _NUM_COLS * _CPI
_L = 16

_mesh = plsc.VectorSubcoreMesh(core_axis_name="c", subcore_axis_name="s")


@functools.partial(
    pl.kernel,
    mesh=_mesh,
    compiler_params=pltpu.CompilerParams(
        use_tc_tiling_on_sc=True, needs_layout_passes=False
    ),
    out_type=jax.ShapeDtypeStruct((_NUM_COLS * _NUM_ROWS * _DIM,), jnp.float32),
    scratch_types=[
        pltpu.VMEM((1, _CHUNK), jnp.int32),
        pltpu.VMEM((1, _CHUNK), jnp.int32),
        pltpu.VMEM((_CHUNK,), jnp.int32),
        pltpu.VMEM((_CHUNK,), jnp.int32),
        pltpu.VMEM((_CHUNK,), jnp.int32),
        pltpu.VMEM((_CHUNK,), jnp.int32),
        pltpu.VMEM((_CHUNK, 2 * _DIM), jnp.float32),
        pltpu.VMEM((_CHUNK, 2 * _DIM), jnp.float32),
        pltpu.VMEM((_CHUNK * _DIM,), jnp.float32),
        pltpu.VMEM((_CHUNK * _DIM,), jnp.float32),
        pltpu.SemaphoreType.DMA((2,)),
        pltpu.SemaphoreType.DMA((2,)),
        pltpu.SemaphoreType.DMA((2,)),
    ],
)
def _gather_kernel(xt_hbm, tbl_hbm, out_hbm, idx_v0, idx_v1, idxh_v0, idxh_v1,
                   par_v0, par_v1, rows_v0, rows_v1, sel_v0, sel_v1,
                   isem, gsem, wsem):
    idx_v = (idx_v0, idx_v1)
    idxh_v = (idxh_v0, idxh_v1)
    par_v = (par_v0, par_v1)
    rows_v = (rows_v0, rows_v1)
    sel_v = (sel_v0, sel_v1)
    wid = lax.axis_index("s") * _NC + lax.axis_index("c")
    i0 = wid * _IPW

    def idx_dma(s, b):
        j = s // _CPI
        base_i = i0 + (s % _CPI) * _CHUNK
        return pltpu.make_async_copy(
            xt_hbm.at[pl.ds(j, 1), pl.ds(base_i, _CHUNK)], idx_v[b],
            isem.at[b]
        )

    def gather(b):
        return pltpu.make_async_copy(
            tbl_hbm.at[idxh_v[b]], rows_v[b], gsem.at[b]
        )

    def writeback(s, b):
        j = s // _CPI
        base_i = i0 + (s % _CPI) * _CHUNK
        return pltpu.make_async_copy(
            sel_v[b],
            out_hbm.at[pl.ds((j * _NUM_ROWS + base_i) * _DIM, _CHUNK * _DIM)],
            wsem.at[b],
        )

    def halve(b):
        def body(k, _):
            v = idx_v[b][0, pl.ds(k * _L, _L)]
            idxh_v[b][pl.ds(k * _L, _L)] = lax.shift_right_logical(v, 1)
            par_v[b][pl.ds(k * _L, _L)] = lax.shift_left(
                lax.bitwise_and(v, 1), 6
            )
            return 0

        lax.fori_loop(0, _CHUNK // _L, body, 0, unroll=4)

    def select(b):
        lanes = lax.iota(jnp.int32, _L)

        def body(k, _):
            kvec = k * _L + lanes
            colbase = par_v[b][pl.ds(k * _L, _L)]
            kbase = kvec * _DIM

            def cbody(cc, _):
                c0 = cc * 8
                for dc in range(8):
                    c = c0 + dc
                    val = plsc.load_gather(rows_v[b], [kvec, colbase + c])
                    plsc.store_scatter(sel_v[b], [kbase + c], val)
                return 0

            lax.fori_loop(0, _DIM // 8, cbody, 0)
            return 0

        lax.fori_loop(0, _CHUNK // _L, body, 0)

    def part_a(s, b, wait_wb, prefetch):
        idx_dma(s, b).wait()
        halve(b)
        gather(b).start()
        if prefetch is not None:
            idx_dma(prefetch, 1 - b).start()
        if wait_wb is not None:
            writeback(wait_wb, b).wait()

    def part_b(sm1, bb):
        gather(bb).wait()
        select(bb)
        writeback(sm1, bb).start()

    idx_dma(0, 0).start()
    part_a(0, 0, None, 1)
    part_a(1, 1, None, 2)
    part_b(0, 0)
    def body(t, carry):
        s0 = 2 * t
        part_a(s0, 0, s0 - 2, s0 + 1)
        part_b(s0 - 1, 1)
        part_a(s0 + 1, 1, s0 - 1, s0 + 2)
        part_b(s0, 0)
        return carry
    lax.fori_loop(1, _NSTEP // 2 - 1, body, 0)
    s0 = _NSTEP - 2
    part_a(s0, 0, s0 - 2, s0 + 1)
    part_b(s0 - 1, 1)
    part_a(s0 + 1, 1, s0 - 1, None)
    part_b(s0, 0)
    part_b(s0 + 1, 1)
    writeback(_NSTEP - 2, 0).wait()
    writeback(_NSTEP - 1, 1).wait()


def kernel(x, weights):
    res = _gather_kernel(x.T, weights.reshape(500000, 2 * _DIM))
    res = res.reshape(_NUM_COLS, _NUM_ROWS, _DIM)
    return jnp.transpose(res, (1, 0, 2))


# speedup vs baseline: 1.0164x; 1.0164x over previous
import functools

import jax
import jax.numpy as jnp
from jax import lax
from jax.experimental import pallas as pl
from jax.experimental.pallas import tpu as pltpu
from jax.experimental.pallas import tpu_sc as plsc

_NUM_ROWS = 16384
_NUM_COLS = 26
_DIM = 64
_V = 1000000
_NC = 2
_NS = 16
_NW = _NC * _NS

_NBLK = (_V + 127) // 128
_TBLK = (_NBLK + _NW - 1) // _NW
_VPAD = _NBLK * 128

_IPW = _NUM_ROWS // _NW
_L = 16

_mesh = plsc.VectorSubcoreMesh(core_axis_name="c", subcore_axis_name="s")


@functools.partial(
    pl.kernel,
    mesh=_mesh,
    compiler_params=pltpu.CompilerParams(use_tc_tiling_on_sc=True),
    out_type=jax.ShapeDtypeStruct((_VPAD // 2, 128), jnp.float32),
    scratch_types=[
        pltpu.VMEM((_DIM, 128), jnp.float32),
        pltpu.VMEM((_DIM, 128), jnp.float32),
        pltpu.VMEM((_DIM, 128), jnp.float32),
        pltpu.SemaphoreType.DMA((3,)),
        pltpu.SemaphoreType.DMA((3,)),
    ],
)
def _pack_kernel(wt_hbm, ptbl_hbm, blk0, blk1, blk2, rsem, wsem):
    wid = lax.axis_index("s") * _NC + lax.axis_index("c")
    blk = (blk0, blk1, blk2)

    def c0_of(t):
        return pl.multiple_of(
            lax.min(wid + _NW * t, _NBLK - 1) * 128, 128
        )

    def rd(t, b):
        return pltpu.make_async_copy(
            wt_hbm.at[:, pl.ds(c0_of(t), 128)], blk[b], rsem.at[b]
        )

    def wr(t, b):
        return pltpu.make_async_copy(
            blk[b], ptbl_hbm.at[pl.ds(lax.div(c0_of(t), 2), _DIM)],
            wsem.at[b]
        )

    def step(t, b, wait_prev, prefetch):
        rd(t, b).wait()
        wr(t, b).start()
        if wait_prev:
            wr(t - 1, (b - 1) % 3).wait()
        if prefetch:
            rd(t + 2, (b + 2) % 3).start()

    rd(0, 0).start()
    rd(1, 1).start()
    step(0, 0, False, True)

    def loop_body(u, carry):
        t = 3 * u
        step(t - 2, 1, True, True)
        step(t - 1, 2, True, True)
        step(t, 0, True, True)
        return carry

    lax.fori_loop(1, 81, loop_body, 0)
    step(_TBLK - 4, (_TBLK - 4) % 3, True, True)
    step(_TBLK - 3, (_TBLK - 3) % 3, True, True)
    step(_TBLK - 2, (_TBLK - 2) % 3, True, False)
    step(_TBLK - 1, (_TBLK - 1) % 3, True, False)
    wr(_TBLK - 1, (_TBLK - 1) % 3).wait()


@functools.partial(
    pl.kernel,
    mesh=_mesh,
    compiler_params=pltpu.CompilerParams(
        use_tc_tiling_on_sc=False, needs_layout_passes=False
    ),
    out_type=jax.ShapeDtypeStruct((_VPAD // 2, 128), jnp.float32),
    scratch_types=[
        pltpu.VMEM((_DIM, 128), jnp.float32),
        pltpu.VMEM((_DIM, 128), jnp.float32),
        pltpu.VMEM((_DIM, 128), jnp.float32),
        pltpu.VMEM((_DIM, 128), jnp.float32),
        pltpu.VMEM((_DIM, 128), jnp.float32),
        pltpu.VMEM((_DIM, 128), jnp.float32),
        pltpu.SemaphoreType.DMA((3,)),
        pltpu.SemaphoreType.DMA((3,)),
    ],
)
def _transpose_kernel(p3_hbm, tbl_hbm, blka0, blka1, blka2, tb0, tb1, tb2,
                      rsem, wsem):
    wid = lax.axis_index("s") * _NC + lax.axis_index("c")
    blk = (blka0, blka1, blka2)
    tb = (tb0, tb1, tb2)
    lanes = lax.iota(jnp.int32, _L)

    def r0_of(t):
        return lax.min(wid + _NW * t, _NBLK - 1) * _DIM

    def rd(t, b):
        return pltpu.make_async_copy(
            p3_hbm.at[pl.ds(r0_of(t), _DIM), :], blk[b], rsem.at[b]
        )

    def wr(t, b):
        return pltpu.make_async_copy(
            tb[b], tbl_hbm.at[pl.ds(r0_of(t), _DIM), :], wsem.at[b]
        )

    srow, scol = [], []
    for g in range(8):
        ri = g * _L + lanes
        srow.append(lax.shift_right_logical(ri, 1))
        scol.append(lax.shift_left(lax.bitwise_and(ri, 1), 6))

    def transpose(b):
        def body(c, carry):
            for g in range(8):
                v = blk[b][c, pl.ds(g * _L, _L)]
                plsc.store_scatter(tb[b], [srow[g], scol[g] + c], v)
            return carry

        lax.fori_loop(0, _DIM, body, 0)

    def step(t, b, wait_prev, prefetch):
        rd(t, b).wait()
        if wait_prev:
            wr(t - 1, (b - 1) % 3).wait()
        transpose(b)
        wr(t, b).start()
        if prefetch:
            rd(t + 2, (b + 2) % 3).start()

    rd(0, 0).start()
    rd(1, 1).start()
    step(0, 0, False, True)

    def loop_body(u, carry):
        t = 3 * u
        step(t - 2, 1, True, True)
        step(t - 1, 2, True, True)
        step(t, 0, True, True)
        return carry

    lax.fori_loop(1, 81, loop_body, 0)
    step(_TBLK - 4, (_TBLK - 4) % 3, True, True)
    step(_TBLK - 3, (_TBLK - 3) % 3, True, True)
    step(_TBLK - 2, (_TBLK - 2) % 3, True, False)
    step(_TBLK - 1, (_TBLK - 1) % 3, True, False)
    wr(_TBLK - 1, (_TBLK - 1) % 3).wait()


@functools.partial(
    pl.kernel,
    mesh=_mesh,
    compiler_params=pltpu.CompilerParams(use_tc_tiling_on_sc=False),
    out_type=jax.ShapeDtypeStruct((_NUM_COLS * _NUM_ROWS, _DIM), jnp.float32),
    scratch_types=[
        pltpu.VMEM((2, _IPW), jnp.int32),
        pltpu.VMEM((2, _IPW, _DIM), jnp.float32),
        pltpu.SemaphoreType.DMA((2,)),
        pltpu.SemaphoreType.DMA((2,)),
        pltpu.SemaphoreType.DMA((2,)),
    ],
)
def _gather_kernel(xt_hbm, tbl_hbm, out_hbm, idx_v, rows_v, isem, gsem, wsem):
    wid = lax.axis_index("s") * _NC + lax.axis_index("c")
    i0 = wid * _IPW

    def idx_dma(j):
        b = j % 2
        return pltpu.make_async_copy(
            xt_hbm.at[j, pl.ds(i0, _IPW)], idx_v.at[b], isem.at[b]
        )

    def gather(j):
        b = j % 2
        return pltpu.make_async_copy(
            tbl_hbm.at[idx_v.at[b]], rows_v.at[b], gsem.at[b]
        )

    def writeback(j):
        b = j % 2
        return pltpu.make_async_copy(
            rows_v.at[b],
            out_hbm.at[pl.ds(j * _NUM_ROWS + i0, _IPW)],
            wsem.at[b],
        )

    idx_dma(0).start()
    for j in range(_NUM_COLS):
        if j >= 2:
            writeback(j - 2).wait()
        idx_dma(j).wait()
        gather(j).start()
        if j + 1 < _NUM_COLS:
            idx_dma(j + 1).start()
        gather(j).wait()
        writeback(j).start()
    writeback(_NUM_COLS - 2).wait()
    writeback(_NUM_COLS - 1).wait()


def kernel(x, weights):
    p3 = _pack_kernel(weights.T)
    ptbl = _transpose_kernel(p3)
    tbl = ptbl.reshape(_VPAD, _DIM)
    res = _gather_kernel(x.T, tbl)
    res = res.reshape(_NUM_COLS, _NUM_ROWS, _DIM)
    return jnp.transpose(res, (1, 0, 2))


# speedup vs baseline: 2.0004x; 1.9680x over previous
import functools

import jax
import jax.numpy as jnp
from jax import lax
from jax.experimental import pallas as pl
from jax.experimental.pallas import tpu as pltpu
from jax.experimental.pallas import tpu_sc as plsc

_NUM_ROWS = 16384
_NUM_COLS = 26
_DIM = 64
_V = 1000000
_NC = 2
_NS = 16
_NW = _NC * _NS

_NBLK = (_V + 127) // 128
_TBLK = (_NBLK + _NW - 1) // _NW
_VPAD = _NBLK * 128

_IPW = _NUM_ROWS // _NW
_L = 16

_mesh = plsc.VectorSubcoreMesh(core_axis_name="c", subcore_axis_name="s")


@functools.partial(
    pl.kernel,
    mesh=_mesh,
    compiler_params=pltpu.CompilerParams(use_tc_tiling_on_sc=True),
    out_type=jax.ShapeDtypeStruct((_VPAD // 2, 128), jnp.float32),
    scratch_types=[
        pltpu.VMEM((_DIM, 128), jnp.float32),
        pltpu.VMEM((_DIM, 128), jnp.float32),
        pltpu.VMEM((_DIM, 128), jnp.float32),
        pltpu.SemaphoreType.DMA((3,)),
        pltpu.SemaphoreType.DMA((3,)),
    ],
)
def _pack_kernel(wt_hbm, ptbl_hbm, blk0, blk1, blk2, rsem, wsem):
    wid = lax.axis_index("s") * _NC + lax.axis_index("c")
    blk = (blk0, blk1, blk2)

    def c0_of(t):
        return pl.multiple_of(
            lax.min(wid + _NW * t, _NBLK - 1) * 128, 128
        )

    def rd(t, b):
        return pltpu.make_async_copy(
            wt_hbm.at[:, pl.ds(c0_of(t), 128)], blk[b], rsem.at[b]
        )

    def wr(t, b):
        return pltpu.make_async_copy(
            blk[b], ptbl_hbm.at[pl.ds(lax.div(c0_of(t), 2), _DIM)],
            wsem.at[b]
        )

    def step(t, b, wait_prev, prefetch):
        rd(t, b).wait()
        wr(t, b).start()
        if wait_prev:
            wr(t - 1, (b - 1) % 3).wait()
        if prefetch:
            rd(t + 2, (b + 2) % 3).start()

    rd(0, 0).start()
    rd(1, 1).start()
    step(0, 0, False, True)

    def loop_body(u, carry):
        t = 3 * u
        step(t - 2, 1, True, True)
        step(t - 1, 2, True, True)
        step(t, 0, True, True)
        return carry

    lax.fori_loop(1, 81, loop_body, 0)
    step(_TBLK - 4, (_TBLK - 4) % 3, True, True)
    step(_TBLK - 3, (_TBLK - 3) % 3, True, True)
    step(_TBLK - 2, (_TBLK - 2) % 3, True, False)
    step(_TBLK - 1, (_TBLK - 1) % 3, True, False)
    wr(_TBLK - 1, (_TBLK - 1) % 3).wait()


@functools.partial(
    pl.kernel,
    mesh=_mesh,
    compiler_params=pltpu.CompilerParams(
        use_tc_tiling_on_sc=False, needs_layout_passes=False
    ),
    out_type=jax.ShapeDtypeStruct((_VPAD // 2, 128), jnp.float32),
    scratch_types=[
        pltpu.VMEM((_DIM, 128), jnp.float32),
        pltpu.VMEM((_DIM, 128), jnp.float32),
        pltpu.VMEM((_DIM, 128), jnp.float32),
        pltpu.VMEM((_DIM, 128), jnp.float32),
        pltpu.VMEM((_DIM, 128), jnp.float32),
        pltpu.VMEM((_DIM, 128), jnp.float32),
        pltpu.SemaphoreType.DMA((3,)),
        pltpu.SemaphoreType.DMA((3,)),
    ],
)
def _transpose_kernel(p3_hbm, tbl_hbm, blka0, blka1, blka2, tb0, tb1, tb2,
                      rsem, wsem):
    wid = lax.axis_index("s") * _NC + lax.axis_index("c")
    blk = (blka0, blka1, blka2)
    tb = (tb0, tb1, tb2)
    lanes = lax.iota(jnp.int32, _L)

    def r0_of(t):
        return lax.min(wid + _NW * t, _NBLK - 1) * _DIM

    def rd(t, b):
        return pltpu.make_async_copy(
            p3_hbm.at[pl.ds(r0_of(t), _DIM), :], blk[b], rsem.at[b]
        )

    def wr(t, b):
        return pltpu.make_async_copy(
            tb[b], tbl_hbm.at[pl.ds(r0_of(t), _DIM), :], wsem.at[b]
        )

    srow, scol = [], []
    for g in range(8):
        ri = g * _L + lanes
        srow.append(lax.shift_right_logical(ri, 1))
        scol.append(lax.shift_left(lax.bitwise_and(ri, 1), 6))

    def transpose(b):
        def body(c, carry):
            for g in range(8):
                v = blk[b][c, pl.ds(g * _L, _L)]
                plsc.store_scatter(tb[b], [srow[g], scol[g] + c], v)
            return carry

        lax.fori_loop(0, _DIM, body, 0)

    def step(t, b, wait_prev, prefetch):
        rd(t, b).wait()
        if wait_prev:
            wr(t - 1, (b - 1) % 3).wait()
        transpose(b)
        wr(t, b).start()
        if prefetch:
            rd(t + 2, (b + 2) % 3).start()

    rd(0, 0).start()
    rd(1, 1).start()
    step(0, 0, False, True)

    def loop_body(u, carry):
        t = 3 * u
        step(t - 2, 1, True, True)
        step(t - 1, 2, True, True)
        step(t, 0, True, True)
        return carry

    lax.fori_loop(1, 81, loop_body, 0)
    step(_TBLK - 4, (_TBLK - 4) % 3, True, True)
    step(_TBLK - 3, (_TBLK - 3) % 3, True, True)
    step(_TBLK - 2, (_TBLK - 2) % 3, True, False)
    step(_TBLK - 1, (_TBLK - 1) % 3, True, False)
    wr(_TBLK - 1, (_TBLK - 1) % 3).wait()


@functools.partial(
    pl.kernel,
    mesh=_mesh,
    compiler_params=pltpu.CompilerParams(use_tc_tiling_on_sc=False),
    out_type=jax.ShapeDtypeStruct((_NUM_COLS * _NUM_ROWS, _DIM), jnp.float32),
    scratch_types=[
        pltpu.VMEM((2, _IPW), jnp.int32),
        pltpu.VMEM((2, _IPW, _DIM), jnp.float32),
        pltpu.SemaphoreType.DMA((2,)),
        pltpu.SemaphoreType.DMA((2,)),
        pltpu.SemaphoreType.DMA((2,)),
    ],
)
def _gather_kernel(xt_hbm, tbl_hbm, out_hbm, idx_v, rows_v, isem, gsem, wsem):
    wid = lax.axis_index("s") * _NC + lax.axis_index("c")
    i0 = wid * _IPW

    def idx_dma(j):
        b = j % 2
        return pltpu.make_async_copy(
            xt_hbm.at[j, pl.ds(i0, _IPW)], idx_v.at[b], isem.at[b]
        )

    def gather(j):
        b = j % 2
        return pltpu.make_async_copy(
            tbl_hbm.at[idx_v.at[b]], rows_v.at[b], gsem.at[b]
        )

    def writeback(j):
        b = j % 2
        return pltpu.make_async_copy(
            rows_v.at[b],
            out_hbm.at[pl.ds(j * _NUM_ROWS + i0, _IPW)],
            wsem.at[b],
        )

    idx_dma(0).start()
    for j in range(_NUM_COLS):
        if j >= 2:
            writeback(j - 2).wait()
        idx_dma(j).wait()
        gather(j).start()
        if j + 1 < _NUM_COLS:
            idx_dma(j + 1).start()
        gather(j).wait()
        writeback(j).start()
    writeback(_NUM_COLS - 2).wait()
    writeback(_NUM_COLS - 1).wait()


def kernel(x, weights):
    res = _gather_kernel(x.T, weights)
    res = res.reshape(_NUM_COLS, _NUM_ROWS, _DIM)
    return jnp.transpose(res, (1, 0, 2))


# speedup vs baseline: 2.0025x; 1.0010x over previous
import functools

import jax
import jax.numpy as jnp
from jax import lax
from jax.experimental import pallas as pl
from jax.experimental.pallas import tpu as pltpu
from jax.experimental.pallas import tpu_sc as plsc

_NUM_ROWS = 16384
_NUM_COLS = 26
_DIM = 64
_V = 1000000
_NC = 2
_NS = 16
_NW = _NC * _NS

_IPW = _NUM_ROWS // _NW

_mesh = plsc.VectorSubcoreMesh(core_axis_name="c", subcore_axis_name="s")


@functools.partial(
    pl.kernel,
    mesh=_mesh,
    compiler_params=pltpu.CompilerParams(use_tc_tiling_on_sc=False),
    out_type=jax.ShapeDtypeStruct((_NUM_COLS * _NUM_ROWS, _DIM), jnp.float32),
    scratch_types=[
        pltpu.VMEM((2, _IPW), jnp.int32),
        pltpu.VMEM((2, _IPW, _DIM), jnp.float32),
        pltpu.SemaphoreType.DMA((2,)),
        pltpu.SemaphoreType.DMA((2,)),
        pltpu.SemaphoreType.DMA((2,)),
    ],
)
def _gather_kernel(xt_hbm, tbl_hbm, out_hbm, idx_v, rows_v, isem, gsem, wsem):
    wid = lax.axis_index("s") * _NC + lax.axis_index("c")
    i0 = wid * _IPW

    def idx_dma(j):
        b = j % 2
        return pltpu.make_async_copy(
            xt_hbm.at[j, pl.ds(i0, _IPW)], idx_v.at[b], isem.at[b]
        )

    def gather(j):
        b = j % 2
        return pltpu.make_async_copy(
            tbl_hbm.at[idx_v.at[b]], rows_v.at[b], gsem.at[b]
        )

    def writeback(j):
        b = j % 2
        return pltpu.make_async_copy(
            rows_v.at[b],
            out_hbm.at[pl.ds(j * _NUM_ROWS + i0, _IPW)],
            wsem.at[b],
        )

    idx_dma(0).start()
    for j in range(_NUM_COLS):
        if j >= 2:
            writeback(j - 2).wait()
        idx_dma(j).wait()
        gather(j).start()
        if j + 1 < _NUM_COLS:
            idx_dma(j + 1).start()
        gather(j).wait()
        writeback(j).start()
    writeback(_NUM_COLS - 2).wait()
    writeback(_NUM_COLS - 1).wait()


def kernel(x, weights):
    res = _gather_kernel(x.T, weights)
    res = res.reshape(_NUM_COLS, _NUM_ROWS, _DIM)
    return jnp.transpose(res, (1, 0, 2))
